# baseline (speedup 1.0000x reference)
import functools

import jax
import jax.numpy as jnp
from jax import lax
from jax.experimental import pallas as pl
from jax.experimental.pallas import tpu as pltpu
from jax.experimental.pallas import tpu_sc as plsc

N = 50000
NP = 50176
CHUNK = 6272
ACC_R = 6304
QPC = 4
E = 150000
EP = 150016
ET = EP // 16
G = ET // 16
PIECES = 2
PE = ET // PIECES
PG = PE // 16
CAP = 4864
K = 128
D = 128
CROWS = CHUNK // D
CR_P = 56
NCNT = (NP // CHUNK) * CR_P


def _seg_body(src_hbm, dst_hbm, x_hbm, sum_hbm, cnt_hbm,
              src_t, dst_t, pos_c, idx_g, idx_b, idx_g2, idx_b2,
              idx56, rows, rows2, oh, sum_acc, cnt_acc, sem, sem2):
    c = lax.axis_index("c")
    t = lax.axis_index("s")

    zero16 = jnp.zeros((16,), jnp.float32)
    one16f = jnp.ones((16,), jnp.float32)
    lane = lax.iota(jnp.int32, 16)

    for j in range(4):
        idx56[pl.ds(j * 16, 16)] = jnp.int32(j * 16) + lane

    for q in range(QPC):
        base = (c * QPC + q) * CHUNK
        qq = c * QPC + q

        def zr(i, _):
            rows[i // 8, pl.ds((i % 8) * 16, 16)] = zero16
            return 0
        lax.fori_loop(0, K * 8, zr, 0)

        def zoh(i, _):
            oh[i // 8, pl.ds((i % 8) * 16, 16)] = zero16
            return 0
        lax.fori_loop(0, 64 * 8, zoh, 0)
        for j in range(3):
            pltpu.sync_copy(rows, sum_acc.at[pl.ds(t * 392 + j * 128, 128)])
        pltpu.sync_copy(rows.at[pl.ds(0, 8)],
                        sum_acc.at[pl.ds(t * 392 + 384, 8)])

        @pl.when(t == 0)
        def _zero_cnt():
            pltpu.sync_copy(rows.at[pl.ds(0, 64)], cnt_acc)
        plsc.subcore_barrier()

        trash = jnp.int32(CAP - 16) + lane
        bufs = ((idx_g, rows, sem), (idx_g2, rows2, sem2))
        ibufs = (idx_b, idx_b2)

        for piece in range(PIECES):
            eoff = t * ET + piece * PE
            pltpu.sync_copy(src_hbm.at[pl.ds(eoff, PE)], src_t)
            pltpu.sync_copy(dst_hbm.at[pl.ds(eoff, PE)], dst_t)

            def cmp_body(i, off_v):
                d = dst_t[pl.ds(i * 16, 16)]
                ld = d - base
                m = ld.astype(jnp.uint32) < jnp.uint32(CHUNK)
                mi = m.astype(jnp.int32)
                pos = jnp.where(m, off_v + plsc.cumsum(mi) - 1, trash)
                plsc.store_scatter(pos_c, [pos], i * 16 + lane)
                return off_v + plsc.all_reduce_population_count(m)
            off_v = lax.fori_loop(0, PG, cmp_body,
                                  jnp.zeros((16,), jnp.int32))
            nmatch = off_v[0]
            nb = (nmatch + (K - 1)) // K

            def build_idx(b, ig, ib):
                for j in range(K // 16):
                    gpos = b * K + j * 16 + lane
                    valid = gpos < nmatch
                    pv = pos_c[pl.ds(b * K + j * 16, 16)]
                    pv = jnp.where(valid, pv, 0)
                    sv = plsc.load_gather(src_t, [pv])
                    dv = plsc.load_gather(dst_t, [pv]) - base
                    ig[pl.ds(j * 16, 16)] = jnp.where(valid, sv, 0)
                    ib[pl.ds(j * 16, 16)] = jnp.where(valid, dv, CHUNK)

            def start_gather(b, u):
                ig, rw, sm = bufs[u]
                build_idx(b, ig, ibufs[u])
                pltpu.async_copy(x_hbm.at[ig], rw, sm)

            def finish_batch(u):
                ig, rw, sm = bufs[u]
                ib = ibufs[u]
                pltpu.make_async_copy(x_hbm.at[ig], rw, sm).wait()
                pltpu.sync_copy(rw, sum_acc.at[ib], add=True)
                for j in range(K // 16):
                    dv = ib[pl.ds(j * 16, 16)]
                    plsc.addupdate_scatter(oh, [dv >> 7, dv & 127], one16f)

            for u in range(2):
                @pl.when(u < nb)
                def _prime(u=u):
                    start_gather(jnp.int32(u), u)

            def pair_body(p, _):
                for u in range(2):
                    b = p * 2 + u

                    @pl.when(b < nb)
                    def _do(b=b, u=u):
                        finish_batch(u)

                        @pl.when(b + 2 < nb)
                        def _next():
                            start_gather(b + 2, u)
                return 0
            lax.fori_loop(0, (nb + 1) // 2, pair_body, 0)

        pltpu.sync_copy(oh, cnt_acc.at[idx56], add=True)

        plsc.subcore_barrier()

        pltpu.sync_copy(sum_acc.at[pl.ds(t * 392, 392)],
                        sum_hbm.at[pl.ds(base + t * 392, 392)])

        @pl.when(t == 0)
        def _cnt_out():
            pltpu.sync_copy(cnt_acc.at[pl.ds(0, CR_P)],
                            cnt_hbm.at[pl.ds(qq * CR_P, CR_P)])
        plsc.subcore_barrier()


def _make_seg_kernel():
    mesh = plsc.VectorSubcoreMesh(core_axis_name="c", subcore_axis_name="s")
    return pl.kernel(
        _seg_body,
        out_type=[
            jax.ShapeDtypeStruct((NP, D), jnp.float32),
            jax.ShapeDtypeStruct((NCNT, D), jnp.float32),
        ],
        mesh=mesh,
        compiler_params=pltpu.CompilerParams(needs_layout_passes=False),
        scratch_types=[
            pltpu.VMEM((PE,), jnp.int32),
            pltpu.VMEM((PE,), jnp.int32),
            pltpu.VMEM((CAP,), jnp.int32),
            pltpu.VMEM((K,), jnp.int32),
            pltpu.VMEM((K,), jnp.int32),
            pltpu.VMEM((K,), jnp.int32),
            pltpu.VMEM((K,), jnp.int32),
            pltpu.VMEM((64,), jnp.int32),
            pltpu.VMEM((K, D), jnp.float32),
            pltpu.VMEM((K, D), jnp.float32),
            pltpu.VMEM((64, D), jnp.float32),
            pltpu.VMEM_SHARED((ACC_R, D), jnp.float32),
            pltpu.VMEM_SHARED((64, D), jnp.float32),
            pltpu.SemaphoreType.DMA,
            pltpu.SemaphoreType.DMA,
        ],
    )


def _tc_body(sg2p, cg2p, ss2p, cs2p, xp, sp2g, cp2g, xg, sp2s, cp2s, xs,
             Wlg2p, Wrg2p, blg2p, Wls2p, Wrs2p, bls2p,
             Wlp2g, Wrp2g, blp2g, Wlp2s, Wrp2s, blp2s,
             Wgw, bgw, Wsw, bsw, out_p, out_g, out_s):
    f32 = jnp.float32

    def mean(s_ref, c_ref):
        c = jnp.maximum(c_ref[...][0], 1.0)
        return s_ref[...] / jnp.transpose(c)

    m1 = mean(sg2p, cg2p)
    m2 = mean(ss2p, cs2p)
    accp = (jnp.dot(m1, Wlg2p[...], preferred_element_type=f32)
            + jnp.dot(m2, Wls2p[...], preferred_element_type=f32)
            + jnp.dot(xp[...], Wrg2p[...] + Wrs2p[...],
                      preferred_element_type=f32)
            + blg2p[...] + bls2p[...])
    out_p[...] = jnp.maximum(accp, 0.0)

    mg = mean(sp2g, cp2g)
    hg = (jnp.dot(mg, Wlp2g[...], preferred_element_type=f32)
          + jnp.dot(xg[...], Wrp2g[...], preferred_element_type=f32)
          + blp2g[...])
    hg = jnp.maximum(hg, 0.0)
    out_g[...] = jnp.dot(hg, Wgw[...], preferred_element_type=f32) + bgw[...]

    ms = mean(sp2s, cp2s)
    hs = (jnp.dot(ms, Wlp2s[...], preferred_element_type=f32)
          + jnp.dot(xs[...], Wrp2s[...], preferred_element_type=f32)
          + blp2s[...])
    hs = jnp.maximum(hs, 0.0)
    out_s[...] = jnp.dot(hs, Wsw[...], preferred_element_type=f32) + bsw[...]


def _run_tc(sg2p, cg2p, ss2p, cs2p, xp, sp2g, cp2g, xg, sp2s, cp2s, xs,
            Wlg2p, Wrg2p, blg2p, Wls2p, Wrs2p, bls2p,
            Wlp2g, Wrp2g, blp2g, Wlp2s, Wrp2s, blp2s, Wgw, bgw, Wsw, bsw):
    nblk = NP // 128
    row = pl.BlockSpec((128, D), lambda i: (i, 0))
    row16 = pl.BlockSpec(
        (1, 1, D), lambda i: ((i // CROWS) * CR_P + i % CROWS, 0, 0))
    wfull = pl.BlockSpec((D, D), lambda i: (0, 0))
    bvec = pl.BlockSpec((1, D), lambda i: (0, 0))
    whead = pl.BlockSpec((D, 1), lambda i: (0, 0))
    bhead = pl.BlockSpec((1, 1), lambda i: (0, 0))
    return pl.pallas_call(
        _tc_body,
        grid=(nblk,),
        in_specs=[row, row16, row, row16, row, row, row16, row, row, row16,
                  row,
                  wfull, wfull, bvec, wfull, wfull, bvec,
                  wfull, wfull, bvec, wfull, wfull, bvec,
                  whead, bhead, whead, bhead],
        out_specs=[row, pl.BlockSpec((128, 1), lambda i: (i, 0)),
                   pl.BlockSpec((128, 1), lambda i: (i, 0))],
        out_shape=[
            jax.ShapeDtypeStruct((NP, D), jnp.float32),
            jax.ShapeDtypeStruct((NP, 1), jnp.float32),
            jax.ShapeDtypeStruct((NP, 1), jnp.float32),
        ],
    )(sg2p, cg2p, ss2p, cs2p, xp, sp2g, cp2g, xg, sp2s, cp2s, xs,
      Wlg2p, Wrg2p, blg2p, Wls2p, Wrs2p, bls2p,
      Wlp2g, Wrp2g, blp2g, Wlp2s, Wrp2s, blp2s, Wgw, bgw, Wsw, bsw)


def _pad_edges(ei):
    src = ei[0].astype(jnp.int32)
    dst = ei[1].astype(jnp.int32)
    pad = EP - E
    src = jnp.concatenate([src, jnp.zeros((pad,), jnp.int32)])
    dst = jnp.concatenate([dst, jnp.full((pad,), jnp.int32(1 << 30))])
    return src, dst


def kernel(x_pfas_sites, x_gw_wells, x_sw_stations,
           edge_index_pfas_to_gw, edge_index_gw_to_pfas,
           edge_index_pfas_to_sw, edge_index_sw_to_pfas,
           Wl_p2g, bl_p2g, Wr_p2g,
           Wl_g2p, bl_g2p, Wr_g2p,
           Wl_p2s, bl_p2s, Wr_p2s,
           Wl_s2p, bl_s2p, Wr_s2p,
           W_gw, b_gw, W_sw, b_sw):
    seg = _make_seg_kernel()

    s_p2g, d_p2g = _pad_edges(edge_index_pfas_to_gw)
    s_g2p, d_g2p = _pad_edges(edge_index_gw_to_pfas)
    s_p2s, d_p2s = _pad_edges(edge_index_pfas_to_sw)
    s_s2p, d_s2p = _pad_edges(edge_index_sw_to_pfas)

    sum_p2g, cnt_p2g = seg(s_p2g, d_p2g, x_pfas_sites)
    sum_g2p, cnt_g2p = seg(s_g2p, d_g2p, x_gw_wells)
    sum_p2s, cnt_p2s = seg(s_p2s, d_p2s, x_pfas_sites)
    sum_s2p, cnt_s2p = seg(s_s2p, d_s2p, x_sw_stations)
    cnt_p2g, cnt_g2p, cnt_p2s, cnt_s2p = (
        x.reshape(NCNT, 1, D)
        for x in (cnt_p2g, cnt_g2p, cnt_p2s, cnt_s2p))

    padr = NP - N
    xp = jnp.pad(x_pfas_sites, ((0, padr), (0, 0)))
    xg = jnp.pad(x_gw_wells, ((0, padr), (0, 0)))
    xs = jnp.pad(x_sw_stations, ((0, padr), (0, 0)))

    out_p, out_g, out_s = _run_tc(
        sum_g2p, cnt_g2p, sum_s2p, cnt_s2p, xp,
        sum_p2g, cnt_p2g, xg, sum_p2s, cnt_p2s, xs,
        Wl_g2p, Wr_g2p, bl_g2p.reshape(1, D),
        Wl_s2p, Wr_s2p, bl_s2p.reshape(1, D),
        Wl_p2g, Wr_p2g, bl_p2g.reshape(1, D),
        Wl_p2s, Wr_p2s, bl_p2s.reshape(1, D),
        W_gw, b_gw.reshape(1, 1), W_sw, b_sw.reshape(1, 1))

    return (out_p[:N], out_g[:N], out_s[:N])


# speedup vs baseline: 1.9203x; 1.9203x over previous
import functools

import jax
import jax.numpy as jnp
from jax import lax
from jax.experimental import pallas as pl
from jax.experimental.pallas import tpu as pltpu
from jax.experimental.pallas import tpu_sc as plsc

N = 50000
NP = 50176
CHUNK = 6272
ACC_R = 6304
QPC = 4
E = 150000
EP = 150016
ET = EP // 16
G = ET // 16
CAP = 9600
K = 64
D = 128
CROWS = CHUNK // D
CR_P = 56
NCNT = (NP // CHUNK) * CR_P


def _seg_body(s1_hbm, d1_hbm, s2_hbm, d2_hbm, s3_hbm, d3_hbm, s4_hbm, d4_hbm,
              x1_hbm, x2_hbm, x3_hbm,
              sum1_hbm, cnt1_hbm, sum2_hbm, cnt2_hbm,
              sum3_hbm, cnt3_hbm, sum4_hbm, cnt4_hbm,
              src_t, dst_t, src_c, ldst_c, idx_g, idx_b, idx_g2, idx_b2,
              idx56, rows, rows2, oh, sum_acc, cnt_acc, sem, sem2):
    c = lax.axis_index("c")
    t = lax.axis_index("s")

    zero16 = jnp.zeros((16,), jnp.float32)
    one16f = jnp.ones((16,), jnp.float32)
    lane = lax.iota(jnp.int32, 16)

    for j in range(4):
        idx56[pl.ds(j * 16, 16)] = jnp.int32(j * 16) + lane

    rels = ((s1_hbm, d1_hbm, x1_hbm, sum1_hbm, cnt1_hbm),
            (s2_hbm, d2_hbm, x2_hbm, sum2_hbm, cnt2_hbm),
            (s3_hbm, d3_hbm, x1_hbm, sum3_hbm, cnt3_hbm),
            (s4_hbm, d4_hbm, x3_hbm, sum4_hbm, cnt4_hbm))

    for src_hbm, dst_hbm, x_hbm, sum_hbm, cnt_hbm in rels:
        pltpu.sync_copy(src_hbm.at[pl.ds(t * ET, ET)], src_t)
        pltpu.sync_copy(dst_hbm.at[pl.ds(t * ET, ET)], dst_t)
        _seg_chunks(c, t, zero16, one16f, lane, x_hbm, sum_hbm, cnt_hbm,
                    src_t, dst_t, src_c, ldst_c, idx_g, idx_b, idx_g2,
                    idx_b2, idx56, rows, rows2, oh, sum_acc, cnt_acc,
                    sem, sem2)


def _seg_chunks(c, t, zero16, one16f, lane, x_hbm, sum_hbm, cnt_hbm,
                src_t, dst_t, src_c, ldst_c, idx_g, idx_b, idx_g2, idx_b2,
                idx56, rows, rows2, oh, sum_acc, cnt_acc, sem, sem2):
    def chunk_body(q, _):
        base = (c * QPC + q) * CHUNK
        qq = c * QPC + q

        def zr(i, _):
            rows[i // 8, pl.ds((i % 8) * 16, 16)] = zero16
            return 0
        lax.fori_loop(0, K * 8, zr, 0)

        def zoh(i, _):
            oh[i // 8, pl.ds((i % 8) * 16, 16)] = zero16
            return 0
        lax.fori_loop(0, K * 8, zoh, 0)
        for j in range(6):
            pltpu.sync_copy(rows, sum_acc.at[pl.ds(t * 392 + j * 64, 64)])
        pltpu.sync_copy(rows.at[pl.ds(0, 8)],
                        sum_acc.at[pl.ds(t * 392 + 384, 8)])

        @pl.when(t == 0)
        def _zero_cnt():
            pltpu.sync_copy(rows, cnt_acc)
        plsc.subcore_barrier()

        trash = jnp.int32(CAP - 16) + lane

        def cmp_body(i, off_v):
            d = dst_t[pl.ds(i * 16, 16)]
            s = src_t[pl.ds(i * 16, 16)]
            ld = d - base
            m = ld.astype(jnp.uint32) < jnp.uint32(CHUNK)
            mi = m.astype(jnp.int32)
            pos = jnp.where(m, off_v + plsc.cumsum(mi) - 1, trash)
            plsc.store_scatter(src_c, [pos], s)
            plsc.store_scatter(ldst_c, [pos], ld)
            return off_v + plsc.all_reduce_population_count(m)
        off_v = lax.fori_loop(0, G, cmp_body, jnp.zeros((16,), jnp.int32))
        nmatch = off_v[0]

        nb = (nmatch + (K - 1)) // K
        bufs = ((idx_g, idx_b, rows, sem), (idx_g2, idx_b2, rows2, sem2))

        def build_idx(b, ig, ib):
            for j in range(K // 16):
                gpos = b * K + j * 16 + lane
                valid = gpos < nmatch
                sv = src_c[pl.ds(b * K + j * 16, 16)]
                dv = ldst_c[pl.ds(b * K + j * 16, 16)]
                ig[pl.ds(j * 16, 16)] = jnp.where(valid, sv, 0)
                ib[pl.ds(j * 16, 16)] = jnp.where(valid, dv, CHUNK)

        def start_gather(b, u):
            ig, ib, rw, sm = bufs[u]
            build_idx(b, ig, ib)
            pltpu.async_copy(x_hbm.at[ig], rw, sm)

        def finish_batch(u):
            ig, ib, rw, sm = bufs[u]
            pltpu.make_async_copy(x_hbm.at[ig], rw, sm).wait()
            pltpu.sync_copy(rw, sum_acc.at[ib], add=True)
            for j in range(K // 16):
                dv = ib[pl.ds(j * 16, 16)]
                plsc.addupdate_scatter(oh, [dv >> 7, dv & 127], one16f)

        for u in range(2):
            @pl.when(u < nb)
            def _prime(u=u):
                start_gather(jnp.int32(u), u)

        def pair_body(p, _):
            for u in range(2):
                b = p * 2 + u

                @pl.when(b < nb)
                def _do(b=b, u=u):
                    finish_batch(u)

                    @pl.when(b + 2 < nb)
                    def _next():
                        start_gather(b + 2, u)
            return 0
        lax.fori_loop(0, (nb + 1) // 2, pair_body, 0)

        pltpu.sync_copy(oh, cnt_acc.at[idx56], add=True)

        plsc.subcore_barrier()

        pltpu.sync_copy(sum_acc.at[pl.ds(t * 392, 392)],
                        sum_hbm.at[pl.ds(base + t * 392, 392)])

        @pl.when(t == 0)
        def _cnt_out():
            pltpu.sync_copy(cnt_acc.at[pl.ds(0, CR_P)],
                            cnt_hbm.at[pl.ds(qq * CR_P, CR_P)])
        plsc.subcore_barrier()
        return 0

    lax.fori_loop(0, QPC, chunk_body, 0)


def _make_seg_kernel():
    mesh = plsc.VectorSubcoreMesh(core_axis_name="c", subcore_axis_name="s")
    return pl.kernel(
        _seg_body,
        out_type=[
            jax.ShapeDtypeStruct((NP, D), jnp.float32),
            jax.ShapeDtypeStruct((NCNT, D), jnp.float32),
        ] * 4,
        mesh=mesh,
        compiler_params=pltpu.CompilerParams(needs_layout_passes=False),
        scratch_types=[
            pltpu.VMEM((ET,), jnp.int32),
            pltpu.VMEM((ET,), jnp.int32),
            pltpu.VMEM((CAP,), jnp.int32),
            pltpu.VMEM((CAP,), jnp.int32),
            pltpu.VMEM((K,), jnp.int32),
            pltpu.VMEM((K,), jnp.int32),
            pltpu.VMEM((K,), jnp.int32),
            pltpu.VMEM((K,), jnp.int32),
            pltpu.VMEM((K,), jnp.int32),
            pltpu.VMEM((K, D), jnp.float32),
            pltpu.VMEM((K, D), jnp.float32),
            pltpu.VMEM((K, D), jnp.float32),
            pltpu.VMEM_SHARED((ACC_R, D), jnp.float32),
            pltpu.VMEM_SHARED((K, D), jnp.float32),
            pltpu.SemaphoreType.DMA,
            pltpu.SemaphoreType.DMA,
        ],
    )


def _tc_body(sg2p, cg2p, ss2p, cs2p, xp, sp2g, cp2g, xg, sp2s, cp2s, xs,
             Wlg2p, Wrg2p, blg2p, Wls2p, Wrs2p, bls2p,
             Wlp2g, Wrp2g, blp2g, Wlp2s, Wrp2s, blp2s,
             Wgw, bgw, Wsw, bsw, out_p, out_g, out_s):
    f32 = jnp.float32

    def mean(s_ref, c_ref):
        c = jnp.maximum(c_ref[...][0], 1.0)
        return s_ref[...] / jnp.transpose(c)

    m1 = mean(sg2p, cg2p)
    m2 = mean(ss2p, cs2p)
    accp = (jnp.dot(m1, Wlg2p[...], preferred_element_type=f32)
            + jnp.dot(m2, Wls2p[...], preferred_element_type=f32)
            + jnp.dot(xp[...], Wrg2p[...] + Wrs2p[...],
                      preferred_element_type=f32)
            + blg2p[...] + bls2p[...])
    out_p[...] = jnp.maximum(accp, 0.0)

    mg = mean(sp2g, cp2g)
    hg = (jnp.dot(mg, Wlp2g[...], preferred_element_type=f32)
          + jnp.dot(xg[...], Wrp2g[...], preferred_element_type=f32)
          + blp2g[...])
    hg = jnp.maximum(hg, 0.0)
    out_g[...] = jnp.dot(hg, Wgw[...], preferred_element_type=f32) + bgw[...]

    ms = mean(sp2s, cp2s)
    hs = (jnp.dot(ms, Wlp2s[...], preferred_element_type=f32)
          + jnp.dot(xs[...], Wrp2s[...], preferred_element_type=f32)
          + blp2s[...])
    hs = jnp.maximum(hs, 0.0)
    out_s[...] = jnp.dot(hs, Wsw[...], preferred_element_type=f32) + bsw[...]


def _run_tc(sg2p, cg2p, ss2p, cs2p, xp, sp2g, cp2g, xg, sp2s, cp2s, xs,
            Wlg2p, Wrg2p, blg2p, Wls2p, Wrs2p, bls2p,
            Wlp2g, Wrp2g, blp2g, Wlp2s, Wrp2s, blp2s, Wgw, bgw, Wsw, bsw):
    nblk = NP // 128
    row = pl.BlockSpec((128, D), lambda i: (i, 0))
    row16 = pl.BlockSpec(
        (1, 1, D), lambda i: ((i // CROWS) * CR_P + i % CROWS, 0, 0))
    wfull = pl.BlockSpec((D, D), lambda i: (0, 0))
    bvec = pl.BlockSpec((1, D), lambda i: (0, 0))
    whead = pl.BlockSpec((D, 1), lambda i: (0, 0))
    bhead = pl.BlockSpec((1, 1), lambda i: (0, 0))
    return pl.pallas_call(
        _tc_body,
        grid=(nblk,),
        in_specs=[row, row16, row, row16, row, row, row16, row, row, row16,
                  row,
                  wfull, wfull, bvec, wfull, wfull, bvec,
                  wfull, wfull, bvec, wfull, wfull, bvec,
                  whead, bhead, whead, bhead],
        out_specs=[row, pl.BlockSpec((128, 1), lambda i: (i, 0)),
                   pl.BlockSpec((128, 1), lambda i: (i, 0))],
        out_shape=[
            jax.ShapeDtypeStruct((NP, D), jnp.float32),
            jax.ShapeDtypeStruct((NP, 1), jnp.float32),
            jax.ShapeDtypeStruct((NP, 1), jnp.float32),
        ],
    )(sg2p, cg2p, ss2p, cs2p, xp, sp2g, cp2g, xg, sp2s, cp2s, xs,
      Wlg2p, Wrg2p, blg2p, Wls2p, Wrs2p, bls2p,
      Wlp2g, Wrp2g, blp2g, Wlp2s, Wrp2s, blp2s, Wgw, bgw, Wsw, bsw)


def _pad_edges(ei):
    src = ei[0].astype(jnp.int32)
    dst = ei[1].astype(jnp.int32)
    pad = EP - E
    src = jnp.concatenate([src, jnp.zeros((pad,), jnp.int32)])
    dst = jnp.concatenate([dst, jnp.full((pad,), jnp.int32(1 << 30))])
    return src, dst


def kernel(x_pfas_sites, x_gw_wells, x_sw_stations,
           edge_index_pfas_to_gw, edge_index_gw_to_pfas,
           edge_index_pfas_to_sw, edge_index_sw_to_pfas,
           Wl_p2g, bl_p2g, Wr_p2g,
           Wl_g2p, bl_g2p, Wr_g2p,
           Wl_p2s, bl_p2s, Wr_p2s,
           Wl_s2p, bl_s2p, Wr_s2p,
           W_gw, b_gw, W_sw, b_sw):
    seg = _make_seg_kernel()

    s_p2g, d_p2g = _pad_edges(edge_index_pfas_to_gw)
    s_g2p, d_g2p = _pad_edges(edge_index_gw_to_pfas)
    s_p2s, d_p2s = _pad_edges(edge_index_pfas_to_sw)
    s_s2p, d_s2p = _pad_edges(edge_index_sw_to_pfas)

    (sum_p2g, cnt_p2g, sum_g2p, cnt_g2p,
     sum_p2s, cnt_p2s, sum_s2p, cnt_s2p) = seg(
        s_p2g, d_p2g, s_g2p, d_g2p, s_p2s, d_p2s, s_s2p, d_s2p,
        x_pfas_sites, x_gw_wells, x_sw_stations)
    cnt_p2g, cnt_g2p, cnt_p2s, cnt_s2p = (
        x.reshape(NCNT, 1, D)
        for x in (cnt_p2g, cnt_g2p, cnt_p2s, cnt_s2p))

    padr = NP - N
    xp = jnp.pad(x_pfas_sites, ((0, padr), (0, 0)))
    xg = jnp.pad(x_gw_wells, ((0, padr), (0, 0)))
    xs = jnp.pad(x_sw_stations, ((0, padr), (0, 0)))

    out_p, out_g, out_s = _run_tc(
        sum_g2p, cnt_g2p, sum_s2p, cnt_s2p, xp,
        sum_p2g, cnt_p2g, xg, sum_p2s, cnt_p2s, xs,
        Wl_g2p, Wr_g2p, bl_g2p.reshape(1, D),
        Wl_s2p, Wr_s2p, bl_s2p.reshape(1, D),
        Wl_p2g, Wr_p2g, bl_p2g.reshape(1, D),
        Wl_p2s, Wr_p2s, bl_p2s.reshape(1, D),
        W_gw, b_gw.reshape(1, 1), W_sw, b_sw.reshape(1, 1))

    return (out_p[:N], out_g[:N], out_s[:N])


# speedup vs baseline: 2.0148x; 1.0492x over previous
import functools

import jax
import jax.numpy as jnp
from jax import lax
from jax.experimental import pallas as pl
from jax.experimental.pallas import tpu as pltpu
from jax.experimental.pallas import tpu_sc as plsc

N = 50000
NP = 50176
CHUNK = 6272
ACC_R = 6304
QPC = 4
E = 150000
EP = 150016
ET = EP // 16
G = ET // 16
CAP = 9600
K = 64
D = 128
CROWS = CHUNK // D
CR_P = 56
NCNT = (NP // CHUNK) * CR_P


def _seg_body(s1_hbm, d1_hbm, s2_hbm, d2_hbm, s3_hbm, d3_hbm, s4_hbm, d4_hbm,
              x1_hbm, x2_hbm, x3_hbm,
              sum1_hbm, cnt1_hbm, sum2_hbm, cnt2_hbm,
              sum3_hbm, cnt3_hbm, sum4_hbm, cnt4_hbm,
              src_t, dst_t, src_c, ldst_c, idx_g, idx_b, idx_g2, idx_b2,
              idx56, rows, rows2, oh, sum_acc, cnt_acc, sem, sem2):
    c = lax.axis_index("c")
    t = lax.axis_index("s")

    zero16 = jnp.zeros((16,), jnp.float32)
    one16f = jnp.ones((16,), jnp.float32)
    lane = lax.iota(jnp.int32, 16)

    for j in range(4):
        idx56[pl.ds(j * 16, 16)] = jnp.int32(j * 16) + lane

    rels = ((s1_hbm, d1_hbm, x1_hbm, sum1_hbm, cnt1_hbm),
            (s2_hbm, d2_hbm, x2_hbm, sum2_hbm, cnt2_hbm),
            (s3_hbm, d3_hbm, x1_hbm, sum3_hbm, cnt3_hbm),
            (s4_hbm, d4_hbm, x3_hbm, sum4_hbm, cnt4_hbm))

    for src_hbm, dst_hbm, x_hbm, sum_hbm, cnt_hbm in rels:
        pltpu.sync_copy(src_hbm.at[pl.ds(t * ET, ET)], src_t)
        pltpu.sync_copy(dst_hbm.at[pl.ds(t * ET, ET)], dst_t)
        _seg_chunks(c, t, zero16, one16f, lane, x_hbm, sum_hbm, cnt_hbm,
                    src_t, dst_t, src_c, ldst_c, idx_g, idx_b, idx_g2,
                    idx_b2, idx56, rows, rows2, oh, sum_acc, cnt_acc,
                    sem, sem2)


def _seg_chunks(c, t, zero16, one16f, lane, x_hbm, sum_hbm, cnt_hbm,
                src_t, dst_t, src_c, ldst_c, idx_g, idx_b, idx_g2, idx_b2,
                idx56, rows, rows2, oh, sum_acc, cnt_acc, sem, sem2):
    def chunk_body(q, _):
        base = (c * QPC + q) * CHUNK
        qq = c * QPC + q

        def zr(i, _):
            rows[i // 8, pl.ds((i % 8) * 16, 16)] = zero16
            return 0
        lax.fori_loop(0, K * 8, zr, 0)

        def zoh(i, _):
            oh[i // 8, pl.ds((i % 8) * 16, 16)] = zero16
            return 0
        lax.fori_loop(0, K * 8, zoh, 0)
        for j in range(6):
            pltpu.sync_copy(rows, sum_acc.at[pl.ds(t * 392 + j * 64, 64)])
        pltpu.sync_copy(rows.at[pl.ds(0, 8)],
                        sum_acc.at[pl.ds(t * 392 + 384, 8)])

        @pl.when(t == 0)
        def _zero_cnt():
            pltpu.sync_copy(rows, cnt_acc)
        plsc.subcore_barrier()

        trash = jnp.int32(CAP - 16) + lane

        def cmp_body(i, off_v):
            d = dst_t[pl.ds(i * 16, 16)]
            s = src_t[pl.ds(i * 16, 16)]
            ld = d - base
            m = ld.astype(jnp.uint32) < jnp.uint32(CHUNK)
            mi = m.astype(jnp.int32)
            pos = jnp.where(m, off_v + plsc.cumsum(mi) - 1, trash)
            plsc.store_scatter(src_c, [pos], s)
            plsc.store_scatter(ldst_c, [pos], ld)
            return off_v + plsc.all_reduce_population_count(m)
        off_v = lax.fori_loop(0, G, cmp_body, jnp.zeros((16,), jnp.int32))
        nmatch = off_v[0]

        nb = (nmatch + (K - 1)) // K
        bufs = ((idx_g, idx_b, rows, sem), (idx_g2, idx_b2, rows2, sem2))

        def build_idx(b, ig, ib):
            for j in range(K // 16):
                gpos = b * K + j * 16 + lane
                valid = gpos < nmatch
                sv = src_c[pl.ds(b * K + j * 16, 16)]
                dv = ldst_c[pl.ds(b * K + j * 16, 16)]
                ig[pl.ds(j * 16, 16)] = jnp.where(valid, sv, 0)
                ib[pl.ds(j * 16, 16)] = jnp.where(valid, dv, CHUNK)

        def start_gather(b, u):
            ig, ib, rw, sm = bufs[u]
            build_idx(b, ig, ib)
            pltpu.async_copy(x_hbm.at[ig], rw, sm)

        def finish_batch(u):
            ig, ib, rw, sm = bufs[u]
            pltpu.make_async_copy(x_hbm.at[ig], rw, sm).wait()
            pltpu.sync_copy(rw, sum_acc.at[ib], add=True)
            for j in range(K // 16):
                dv = ib[pl.ds(j * 16, 16)]
                plsc.addupdate_scatter(oh, [dv >> 7, dv & 127], one16f)

        for u in range(2):
            @pl.when(u < nb)
            def _prime(u=u):
                start_gather(jnp.int32(u), u)

        def pair_body(p, _):
            for u in range(2):
                b = p * 2 + u

                @pl.when(b < nb)
                def _do(b=b, u=u):
                    finish_batch(u)

                    @pl.when(b + 2 < nb)
                    def _next():
                        start_gather(b + 2, u)
            return 0
        lax.fori_loop(0, (nb + 1) // 2, pair_body, 0)

        pltpu.sync_copy(oh, cnt_acc.at[idx56], add=True)

        plsc.subcore_barrier()

        pltpu.sync_copy(sum_acc.at[pl.ds(t * 392, 392)],
                        sum_hbm.at[pl.ds(base + t * 392, 392)])

        @pl.when(t == 0)
        def _cnt_out():
            pltpu.sync_copy(cnt_acc.at[pl.ds(0, CR_P)],
                            cnt_hbm.at[pl.ds(qq * CR_P, CR_P)])
        plsc.subcore_barrier()
        return 0

    lax.fori_loop(0, QPC, chunk_body, 0)


def _make_seg_kernel():
    mesh = plsc.VectorSubcoreMesh(core_axis_name="c", subcore_axis_name="s")
    return pl.kernel(
        _seg_body,
        out_type=[
            jax.ShapeDtypeStruct((NP, D), jnp.float32),
            jax.ShapeDtypeStruct((NCNT, D), jnp.float32),
        ] * 4,
        mesh=mesh,
        compiler_params=pltpu.CompilerParams(needs_layout_passes=False),
        scratch_types=[
            pltpu.VMEM((ET,), jnp.int32),
            pltpu.VMEM((ET,), jnp.int32),
            pltpu.VMEM((CAP,), jnp.int32),
            pltpu.VMEM((CAP,), jnp.int32),
            pltpu.VMEM((K,), jnp.int32),
            pltpu.VMEM((K,), jnp.int32),
            pltpu.VMEM((K,), jnp.int32),
            pltpu.VMEM((K,), jnp.int32),
            pltpu.VMEM((K,), jnp.int32),
            pltpu.VMEM((K, D), jnp.float32),
            pltpu.VMEM((K, D), jnp.float32),
            pltpu.VMEM((K, D), jnp.float32),
            pltpu.VMEM_SHARED((ACC_R, D), jnp.float32),
            pltpu.VMEM_SHARED((K, D), jnp.float32),
            pltpu.SemaphoreType.DMA,
            pltpu.SemaphoreType.DMA,
        ],
    )


def _tc_body(sg2p, cg2p, ss2p, cs2p, xp, sp2g, cp2g, xg, sp2s, cp2s, xs,
             Wlg2p, Wrg2p, blg2p, Wls2p, Wrs2p, bls2p,
             Wlp2g, Wrp2g, blp2g, Wlp2s, Wrp2s, blp2s,
             Wgw, bgw, Wsw, bsw, out_p, out_g, out_s):
    f32 = jnp.float32

    def mean(s_ref, c_ref):
        c = jnp.maximum(c_ref[...][0], 1.0)
        return s_ref[...] / jnp.transpose(c)

    m1 = mean(sg2p, cg2p)
    m2 = mean(ss2p, cs2p)
    accp = (jnp.dot(m1, Wlg2p[...], preferred_element_type=f32)
            + jnp.dot(m2, Wls2p[...], preferred_element_type=f32)
            + jnp.dot(xp[...], Wrg2p[...] + Wrs2p[...],
                      preferred_element_type=f32)
            + blg2p[...] + bls2p[...])
    out_p[...] = jnp.maximum(accp, 0.0)

    mg = mean(sp2g, cp2g)
    hg = (jnp.dot(mg, Wlp2g[...], preferred_element_type=f32)
          + jnp.dot(xg[...], Wrp2g[...], preferred_element_type=f32)
          + blp2g[...])
    hg = jnp.maximum(hg, 0.0)
    out_g[...] = jnp.dot(hg, Wgw[...], preferred_element_type=f32) + bgw[...]

    ms = mean(sp2s, cp2s)
    hs = (jnp.dot(ms, Wlp2s[...], preferred_element_type=f32)
          + jnp.dot(xs[...], Wrp2s[...], preferred_element_type=f32)
          + blp2s[...])
    hs = jnp.maximum(hs, 0.0)
    out_s[...] = jnp.dot(hs, Wsw[...], preferred_element_type=f32) + bsw[...]


def _run_tc(sg2p, cg2p, ss2p, cs2p, xp, sp2g, cp2g, xg, sp2s, cp2s, xs,
            Wlg2p, Wrg2p, blg2p, Wls2p, Wrs2p, bls2p,
            Wlp2g, Wrp2g, blp2g, Wlp2s, Wrp2s, blp2s, Wgw, bgw, Wsw, bsw):
    nblk = (N + 127) // 128
    row = pl.BlockSpec((128, D), lambda i: (i, 0))
    row16 = pl.BlockSpec(
        (1, 1, D), lambda i: ((i // CROWS) * CR_P + i % CROWS, 0, 0))
    wfull = pl.BlockSpec((D, D), lambda i: (0, 0))
    bvec = pl.BlockSpec((1, D), lambda i: (0, 0))
    whead = pl.BlockSpec((D, 1), lambda i: (0, 0))
    bhead = pl.BlockSpec((1, 1), lambda i: (0, 0))
    return pl.pallas_call(
        _tc_body,
        grid=(nblk,),
        in_specs=[row, row16, row, row16, row, row, row16, row, row, row16,
                  row,
                  wfull, wfull, bvec, wfull, wfull, bvec,
                  wfull, wfull, bvec, wfull, wfull, bvec,
                  whead, bhead, whead, bhead],
        out_specs=[row, pl.BlockSpec((128, 1), lambda i: (i, 0)),
                   pl.BlockSpec((128, 1), lambda i: (i, 0))],
        out_shape=[
            jax.ShapeDtypeStruct((N, D), jnp.float32),
            jax.ShapeDtypeStruct((N, 1), jnp.float32),
            jax.ShapeDtypeStruct((N, 1), jnp.float32),
        ],
    )(sg2p, cg2p, ss2p, cs2p, xp, sp2g, cp2g, xg, sp2s, cp2s, xs,
      Wlg2p, Wrg2p, blg2p, Wls2p, Wrs2p, bls2p,
      Wlp2g, Wrp2g, blp2g, Wlp2s, Wrp2s, blp2s, Wgw, bgw, Wsw, bsw)


def _pad_edges(ei):
    src = ei[0].astype(jnp.int32)
    dst = ei[1].astype(jnp.int32)
    pad = EP - E
    src = jnp.concatenate([src, jnp.zeros((pad,), jnp.int32)])
    dst = jnp.concatenate([dst, jnp.full((pad,), jnp.int32(1 << 30))])
    return src, dst


def kernel(x_pfas_sites, x_gw_wells, x_sw_stations,
           edge_index_pfas_to_gw, edge_index_gw_to_pfas,
           edge_index_pfas_to_sw, edge_index_sw_to_pfas,
           Wl_p2g, bl_p2g, Wr_p2g,
           Wl_g2p, bl_g2p, Wr_g2p,
           Wl_p2s, bl_p2s, Wr_p2s,
           Wl_s2p, bl_s2p, Wr_s2p,
           W_gw, b_gw, W_sw, b_sw):
    seg = _make_seg_kernel()

    s_p2g, d_p2g = _pad_edges(edge_index_pfas_to_gw)
    s_g2p, d_g2p = _pad_edges(edge_index_gw_to_pfas)
    s_p2s, d_p2s = _pad_edges(edge_index_pfas_to_sw)
    s_s2p, d_s2p = _pad_edges(edge_index_sw_to_pfas)

    (sum_p2g, cnt_p2g, sum_g2p, cnt_g2p,
     sum_p2s, cnt_p2s, sum_s2p, cnt_s2p) = seg(
        s_p2g, d_p2g, s_g2p, d_g2p, s_p2s, d_p2s, s_s2p, d_s2p,
        x_pfas_sites, x_gw_wells, x_sw_stations)
    cnt_p2g, cnt_g2p, cnt_p2s, cnt_s2p = (
        x.reshape(NCNT, 1, D)
        for x in (cnt_p2g, cnt_g2p, cnt_p2s, cnt_s2p))

    xp, xg, xs = x_pfas_sites, x_gw_wells, x_sw_stations

    out_p, out_g, out_s = _run_tc(
        sum_g2p, cnt_g2p, sum_s2p, cnt_s2p, xp,
        sum_p2g, cnt_p2g, xg, sum_p2s, cnt_p2s, xs,
        Wl_g2p, Wr_g2p, bl_g2p.reshape(1, D),
        Wl_s2p, Wr_s2p, bl_s2p.reshape(1, D),
        Wl_p2g, Wr_p2g, bl_p2g.reshape(1, D),
        Wl_p2s, Wr_p2s, bl_p2s.reshape(1, D),
        W_gw, b_gw.reshape(1, 1), W_sw, b_sw.reshape(1, 1))

    return (out_p, out_g, out_s)
